# baseline (speedup 1.0000x reference)
import jax
import jax.numpy as jnp
from jax.experimental import pallas as pl
from jax.experimental.pallas import tpu as pltpu

_BATCH = 4096
_TILE = 512
_NCB = 4
_CBS = 256
_CBD = 64
_BETA = 0.25
_ENC = [768, 512, 256, 128, 64]

_BF = jnp.bfloat16
_F32 = jnp.float32


def _dot(a, b):
    return jax.lax.dot_general(a.astype(_BF), b, (((1,), (0,)), ((), ())),
                               preferred_element_type=_F32)


def _fwd_pair(xs, aWs, abs_, eWss, ebss, dWss, dbss, zWs, zbs, cb3s, cbTs):
    m = len(xs)
    T = xs[0].shape[0]
    hs = [_dot(xs[j], aWs[j]) + abs_[j] for j in range(m)]
    n = len(eWss[0])
    for i in range(n):
        hs = [_dot(hs[j], eWss[j][i]) + ebss[j][i] for j in range(m)]
        if i < n - 1:
            hs = [jnp.maximum(h, 0.0) for h in hs]
    rs = list(hs)
    xqs = [jnp.zeros_like(h) for h in hs]
    ssq_rows = [jnp.zeros((T, 1), dtype=_F32) for _ in range(m)]
    idxs = [[], []]
    colf = jax.lax.broadcasted_iota(jnp.int32, (T, _CBS), 1).astype(_F32)
    for l in range(_NCB):
        cblT = [cbTs[j][l] for j in range(m)]
        cbl3 = [cb3s[j][l] for j in range(m)]
        cb2 = [jnp.sum(c * c, axis=0, keepdims=True) for c in cblT]
        r2 = [jnp.sum(r * r, axis=1, keepdims=True) for r in rs]
        if l > 0:
            ssq_rows = [ssq_rows[j] + r2[j] for j in range(m)]
        dm2 = [_dot(rs[j] * (-2.0), cblT[j].astype(_BF)) for j in range(m)]
        d = [(r2[j] + dm2[j]) + cb2[j] for j in range(m)]
        dmin = [jnp.min(dj, axis=1, keepdims=True) for dj in d]
        idxf = [jnp.min(jnp.where(d[j] == dmin[j], colf, float(_CBS)), axis=1,
                        keepdims=True) for j in range(m)]
        oh = [(colf == i_).astype(_BF) for i_ in idxf]
        q3 = [jax.lax.dot_general(oh[j], cbl3[j], (((1,), (0,)), ((), ())),
                                  preferred_element_type=_F32)
              for j in range(m)]
        q = [(q3j[:, 0:_CBD] + q3j[:, _CBD:2 * _CBD]) + q3j[:, 2 * _CBD:]
             for q3j in q3]
        rs = [rs[j] - q[j] for j in range(m)]
        xqs = [xqs[j] + q[j] for j in range(m)]
        for j in range(m):
            idxs[j].append(idxf[j].astype(jnp.int32))
    ssq_rows = [ssq_rows[j] + jnp.sum(rs[j] * rs[j], axis=1, keepdims=True)
                for j in range(m)]
    ssqs = [jnp.sum(s, keepdims=True).reshape(1, 1) for s in ssq_rows]
    hs = xqs
    for i in range(n):
        hs = [_dot(hs[j], dWss[j][i]) + dbss[j][i] for j in range(m)]
        if i < n - 1:
            hs = [jnp.maximum(h, 0.0) for h in hs]
    outs = [_dot(hs[j], zWs[j]) + zbs[j] for j in range(m)]
    es = [outs[j] - xs[j] for j in range(m)]
    recs = [jnp.sum(e * e, keepdims=True).reshape(1, 1) for e in es]
    inds = [jnp.concatenate(ix, axis=1) for ix in idxs]
    return outs, recs, ssqs, inds


def _kernel_body(*refs):
    (xt_ref, xi_ref, taW, tab, iaW, iab) = refs[0:6]
    teW = refs[6:10]
    teb = refs[10:14]
    ieW = refs[14:18]
    ieb = refs[18:22]
    tdW = refs[22:26]
    tdb = refs[26:30]
    idW = refs[30:34]
    idb = refs[34:38]
    (tzW, tzb, izW, izb, tcb3, tcbT, icb3, icbT) = refs[38:46]
    (out_t_ref, out_i_ref, total_ref, ind_t_ref, ind_i_ref) = refs[46:51]

    outs, recs, ssqs, inds = _fwd_pair(
        [xt_ref[...], xi_ref[...]],
        [taW[...], iaW[...]], [tab[...], iab[...]],
        [[w[...] for w in teW], [w[...] for w in ieW]],
        [[b[...] for b in teb], [b[...] for b in ieb]],
        [[w[...] for w in tdW], [w[...] for w in idW]],
        [[b[...] for b in tdb], [b[...] for b in idb]],
        [tzW[...], izW[...]], [tzb[...], izb[...]],
        [tcb3, icb3], [tcbT, icbT])
    out_t, out_i = outs
    rec_t, rec_i = recs
    ssq_t, ssq_i = ssqs
    ind_t, ind_i = inds

    out_t_ref[...] = out_t
    out_i_ref[...] = out_i
    ind_t_ref[...] = ind_t
    ind_i_ref[...] = ind_i

    c_rec = 1.0 / (_BATCH * _ENC[0])
    c_q = (1.0 + _BETA) / (_NCB * _BATCH * _CBD)
    total_ref[...] = ((rec_t + rec_i) * c_rec
                      + (ssq_t + ssq_i) * c_q).reshape(1, 1, 1)


def _full_spec(shape):
    nd = len(shape)
    return pl.BlockSpec(shape, lambda i, _nd=nd: (0,) * _nd)


def _build_call():
    nt = _BATCH // _TILE
    row_spec = pl.BlockSpec((_TILE, _ENC[0]), lambda i: (i, 0))
    ind_spec = pl.BlockSpec((_TILE, _NCB), lambda i: (i, 0))

    in_specs = [row_spec, row_spec]
    in_specs += [_full_spec((768, 768)), _full_spec((1, 768)),
                 _full_spec((768, 768)), _full_spec((1, 768))]
    enc_w_shapes = [(_ENC[i], _ENC[i + 1]) for i in range(4)]
    dec_w_shapes = [(_ENC[4 - i], _ENC[3 - i]) for i in range(4)]
    enc_b_shapes = [(1, _ENC[i + 1]) for i in range(4)]
    dec_b_shapes = [(1, _ENC[3 - i]) for i in range(4)]
    for shapes in (enc_w_shapes, enc_b_shapes, enc_w_shapes, enc_b_shapes,
                   dec_w_shapes, dec_b_shapes, dec_w_shapes, dec_b_shapes):
        in_specs += [_full_spec(s) for s in shapes]
    in_specs += [_full_spec((768, 768)), _full_spec((1, 768)),
                 _full_spec((768, 768)), _full_spec((1, 768))]
    in_specs += [_full_spec((_NCB, _CBS, 3 * _CBD)), _full_spec((_NCB, _CBD, _CBS)),
                 _full_spec((_NCB, _CBS, 3 * _CBD)), _full_spec((_NCB, _CBD, _CBS))]

    out_specs = [row_spec, row_spec,
                 pl.BlockSpec((1, 1, 1), lambda i: (i, 0, 0)),
                 ind_spec, ind_spec]
    out_shape = [
        jax.ShapeDtypeStruct((_BATCH, 768), _F32),
        jax.ShapeDtypeStruct((_BATCH, 768), _F32),
        jax.ShapeDtypeStruct((nt, 1, 1), _F32),
        jax.ShapeDtypeStruct((_BATCH, _NCB), jnp.int32),
        jax.ShapeDtypeStruct((_BATCH, _NCB), jnp.int32),
    ]
    return pl.pallas_call(
        _kernel_body,
        grid=(nt,),
        in_specs=in_specs,
        out_specs=out_specs,
        out_shape=out_shape,
        compiler_params=pltpu.CompilerParams(
            dimension_semantics=("parallel",)),
    )


def _cb_components(cb):
    hi = cb.astype(_BF)
    rem = cb - hi.astype(_F32)
    mid = rem.astype(_BF)
    lo = (rem - mid.astype(_F32)).astype(_BF)
    return jnp.concatenate([hi, mid, lo], axis=2)


def kernel(text_x, image_x, ta_enc_W, ta_enc_b, ia_enc_W, ia_enc_b,
           te_Ws, te_bs, ie_Ws, ie_bs, td_Ws, td_bs, id_Ws, id_bs,
           ta_dec_W, ta_dec_b, ia_dec_W, ia_dec_b, text_cb, image_cb):
    r2 = lambda b: b.reshape(1, -1)
    w = lambda W: W.astype(_BF)
    args = [text_x, image_x, w(ta_enc_W), r2(ta_enc_b), w(ia_enc_W), r2(ia_enc_b)]
    args += [w(W) for W in te_Ws] + [r2(b) for b in te_bs]
    args += [w(W) for W in ie_Ws] + [r2(b) for b in ie_bs]
    args += [w(W) for W in td_Ws] + [r2(b) for b in td_bs]
    args += [w(W) for W in id_Ws] + [r2(b) for b in id_bs]
    args += [w(ta_dec_W), r2(ta_dec_b), w(ia_dec_W), r2(ia_dec_b)]
    args += [_cb_components(text_cb), jnp.swapaxes(text_cb, 1, 2),
             _cb_components(image_cb), jnp.swapaxes(image_cb, 1, 2)]
    out_t, out_i, totals, ind_t, ind_i = _build_call()(*args)
    return (out_t, out_i, jnp.sum(totals).reshape(()), ind_t, ind_i)


# speedup vs baseline: 1.1913x; 1.1913x over previous
import jax
import jax.numpy as jnp
from jax.experimental import pallas as pl
from jax.experimental.pallas import tpu as pltpu

_BATCH = 4096
_TILE = 512
_NCB = 4
_CBS = 256
_CBD = 64
_BETA = 0.25
_ENC = [768, 512, 256, 128, 64]

_BF = jnp.bfloat16
_F32 = jnp.float32

_ENC_W_SHAPES = [(_ENC[i], _ENC[i + 1]) for i in range(4)]
_DEC_W_SHAPES = [(_ENC[4 - i], _ENC[3 - i]) for i in range(4)]
_W_SHAPES = ([(768, 768), (768, 768)] + _ENC_W_SHAPES + _ENC_W_SHAPES
             + _DEC_W_SHAPES + _DEC_W_SHAPES + [(768, 768), (768, 768)])


def _dot(a, b):
    return jax.lax.dot_general(a.astype(_BF), b, (((1,), (0,)), ((), ())),
                               preferred_element_type=_F32)


def _fwd_pair(xs, aWs, abs_, eWss, ebss, dWss, dbss, zWs, zbs,
              cb3s, cbTbfs, cb2s):
    m = len(xs)
    T = xs[0].shape[0]
    hs = [_dot(xs[j], aWs[j]) + abs_[j] for j in range(m)]
    n = len(eWss[0])
    for i in range(n):
        hs = [_dot(hs[j], eWss[j][i]) + ebss[j][i] for j in range(m)]
        if i < n - 1:
            hs = [jnp.maximum(h, 0.0) for h in hs]
    rs = list(hs)
    xqs = [jnp.zeros_like(h) for h in hs]
    ssq_rows = [jnp.zeros((T, 1), dtype=_F32) for _ in range(m)]
    idxs = [[], []]
    colf = jax.lax.broadcasted_iota(jnp.int32, (T, _CBS), 1).astype(_F32)
    for l in range(_NCB):
        cb2 = [cb2s[j][l] for j in range(m)]
        r2 = [jnp.sum(r * r, axis=1, keepdims=True) for r in rs]
        if l > 0:
            ssq_rows = [ssq_rows[j] + r2[j] for j in range(m)]
        dm2 = [_dot(rs[j] * (-2.0), cbTbfs[j][l]) for j in range(m)]
        d = [(r2[j] + dm2[j]) + cb2[j] for j in range(m)]
        dmin = [jnp.min(dj, axis=1, keepdims=True) for dj in d]
        idxf = [jnp.min(jnp.where(d[j] == dmin[j], colf, float(_CBS)), axis=1,
                        keepdims=True) for j in range(m)]
        oh = [(colf == i_).astype(_BF) for i_ in idxf]
        q3 = [jax.lax.dot_general(oh[j], cb3s[j][l], (((1,), (0,)), ((), ())),
                                  preferred_element_type=_F32)
              for j in range(m)]
        q = [(q3j[:, 0:_CBD] + q3j[:, _CBD:2 * _CBD]) + q3j[:, 2 * _CBD:]
             for q3j in q3]
        rs = [rs[j] - q[j] for j in range(m)]
        xqs = [xqs[j] + q[j] for j in range(m)]
        for j in range(m):
            idxs[j].append(idxf[j].astype(jnp.int32))
    ssq_rows = [ssq_rows[j] + jnp.sum(rs[j] * rs[j], axis=1, keepdims=True)
                for j in range(m)]
    ssqs = [jnp.sum(s, keepdims=True).reshape(1, 1) for s in ssq_rows]
    hs = xqs
    for i in range(n):
        hs = [_dot(hs[j], dWss[j][i]) + dbss[j][i] for j in range(m)]
        if i < n - 1:
            hs = [jnp.maximum(h, 0.0) for h in hs]
    outs = [_dot(hs[j], zWs[j]) + zbs[j] for j in range(m)]
    es = [outs[j] - xs[j] for j in range(m)]
    recs = [jnp.sum(e * e, keepdims=True).reshape(1, 1) for e in es]
    inds = [jnp.concatenate(ix, axis=1) for ix in idxs]
    return outs, recs, ssqs, inds


def _kernel_body(*refs):
    (xt_ref, xi_ref, taW, tab, iaW, iab) = refs[0:6]
    teW = refs[6:10]
    teb = refs[10:14]
    ieW = refs[14:18]
    ieb = refs[18:22]
    tdW = refs[22:26]
    tdb = refs[26:30]
    idW = refs[30:34]
    idb = refs[34:38]
    (tzW, tzb, izW, izb, tcb, tcbT, icb, icbT) = refs[38:46]
    (out_t_ref, out_i_ref, total_ref, ind_t_ref, ind_i_ref) = refs[46:51]
    wscr = refs[51:71]
    (scb3_t, scb3_i, scbT_t, scbT_i, scb2_t, scb2_i) = refs[71:77]

    w_in = [taW, iaW] + list(teW) + list(ieW) + list(tdW) + list(idW) + [tzW, izW]

    i = pl.program_id(0)

    @pl.when(i == 0)
    def _():
        for w_ref, s_ref in zip(w_in, wscr):
            s_ref[...] = w_ref[...].astype(_BF)
        for cb_ref, cbT_ref, s3, sT, s2 in (
                (tcb, tcbT, scb3_t, scbT_t, scb2_t),
                (icb, icbT, scb3_i, scbT_i, scb2_i)):
            for l in range(_NCB):
                c = cb_ref[l]
                hi = c.astype(_BF)
                rem = c - hi.astype(_F32)
                mid = rem.astype(_BF)
                lo = (rem - mid.astype(_F32)).astype(_BF)
                s3[l, :, 0:_CBD] = hi
                s3[l, :, _CBD:2 * _CBD] = mid
                s3[l, :, 2 * _CBD:3 * _CBD] = lo
                ct = cbT_ref[l]
                sT[l] = ct.astype(_BF)
                s2[l] = jnp.sum(ct * ct, axis=0, keepdims=True)

    sw = [s[...] for s in wscr]
    staW, siaW = sw[0], sw[1]
    steW, sieW = sw[2:6], sw[6:10]
    stdW, sidW = sw[10:14], sw[14:18]
    stzW, sizW = sw[18], sw[19]

    outs, recs, ssqs, inds = _fwd_pair(
        [xt_ref[...], xi_ref[...]],
        [staW, siaW], [tab[...], iab[...]],
        [steW, sieW],
        [[b[...] for b in teb], [b[...] for b in ieb]],
        [stdW, sidW],
        [[b[...] for b in tdb], [b[...] for b in idb]],
        [stzW, sizW], [tzb[...], izb[...]],
        [[scb3_t[l] for l in range(_NCB)], [scb3_i[l] for l in range(_NCB)]],
        [[scbT_t[l] for l in range(_NCB)], [scbT_i[l] for l in range(_NCB)]],
        [[scb2_t[l] for l in range(_NCB)], [scb2_i[l] for l in range(_NCB)]])
    out_t, out_i = outs
    rec_t, rec_i = recs
    ssq_t, ssq_i = ssqs
    ind_t, ind_i = inds

    out_t_ref[...] = out_t
    out_i_ref[...] = out_i
    ind_t_ref[...] = ind_t
    ind_i_ref[...] = ind_i

    c_rec = 1.0 / (_BATCH * _ENC[0])
    c_q = (1.0 + _BETA) / (_NCB * _BATCH * _CBD)
    partial = (rec_t + rec_i) * c_rec + (ssq_t + ssq_i) * c_q

    @pl.when(i == 0)
    def _():
        total_ref[...] = partial

    @pl.when(i > 0)
    def _():
        total_ref[...] = total_ref[...] + partial


def _full_spec(shape):
    nd = len(shape)
    return pl.BlockSpec(shape, lambda i, _nd=nd: (0,) * _nd)


def _build_call():
    nt = _BATCH // _TILE
    row_spec = pl.BlockSpec((_TILE, _ENC[0]), lambda i: (i, 0))
    ind_spec = pl.BlockSpec((_TILE, _NCB), lambda i: (i, 0))

    in_specs = [row_spec, row_spec]
    in_specs += [_full_spec((768, 768)), _full_spec((1, 768)),
                 _full_spec((768, 768)), _full_spec((1, 768))]
    enc_b_shapes = [(1, _ENC[i + 1]) for i in range(4)]
    dec_b_shapes = [(1, _ENC[3 - i]) for i in range(4)]
    for shapes in (_ENC_W_SHAPES, enc_b_shapes, _ENC_W_SHAPES, enc_b_shapes,
                   _DEC_W_SHAPES, dec_b_shapes, _DEC_W_SHAPES, dec_b_shapes):
        in_specs += [_full_spec(s) for s in shapes]
    in_specs += [_full_spec((768, 768)), _full_spec((1, 768)),
                 _full_spec((768, 768)), _full_spec((1, 768))]
    in_specs += [_full_spec((_NCB, _CBS, _CBD)), _full_spec((_NCB, _CBD, _CBS)),
                 _full_spec((_NCB, _CBS, _CBD)), _full_spec((_NCB, _CBD, _CBS))]

    out_specs = [row_spec, row_spec,
                 pl.BlockSpec((1, 1), lambda i: (0, 0)),
                 ind_spec, ind_spec]
    out_shape = [
        jax.ShapeDtypeStruct((_BATCH, 768), _F32),
        jax.ShapeDtypeStruct((_BATCH, 768), _F32),
        jax.ShapeDtypeStruct((1, 1), _F32),
        jax.ShapeDtypeStruct((_BATCH, _NCB), jnp.int32),
        jax.ShapeDtypeStruct((_BATCH, _NCB), jnp.int32),
    ]
    scratch = [pltpu.VMEM(s, _BF) for s in _W_SHAPES]
    scratch += [pltpu.VMEM((_NCB, _CBS, 3 * _CBD), _BF),
                pltpu.VMEM((_NCB, _CBS, 3 * _CBD), _BF),
                pltpu.VMEM((_NCB, _CBD, _CBS), _BF),
                pltpu.VMEM((_NCB, _CBD, _CBS), _BF),
                pltpu.VMEM((_NCB, 1, _CBS), _F32),
                pltpu.VMEM((_NCB, 1, _CBS), _F32)]
    return pl.pallas_call(
        _kernel_body,
        grid=(nt,),
        in_specs=in_specs,
        out_specs=out_specs,
        out_shape=out_shape,
        scratch_shapes=scratch,
    )


def kernel(text_x, image_x, ta_enc_W, ta_enc_b, ia_enc_W, ia_enc_b,
           te_Ws, te_bs, ie_Ws, ie_bs, td_Ws, td_bs, id_Ws, id_bs,
           ta_dec_W, ta_dec_b, ia_dec_W, ia_dec_b, text_cb, image_cb):
    r2 = lambda b: b.reshape(1, -1)
    args = [text_x, image_x, ta_enc_W, r2(ta_enc_b), ia_enc_W, r2(ia_enc_b)]
    args += list(te_Ws) + [r2(b) for b in te_bs]
    args += list(ie_Ws) + [r2(b) for b in ie_bs]
    args += list(td_Ws) + [r2(b) for b in td_bs]
    args += list(id_Ws) + [r2(b) for b in id_bs]
    args += [ta_dec_W, r2(ta_dec_b), ia_dec_W, r2(ia_dec_b)]
    args += [text_cb, jnp.swapaxes(text_cb, 1, 2),
             image_cb, jnp.swapaxes(image_cb, 1, 2)]
    out_t, out_i, total, ind_t, ind_i = _build_call()(*args)
    return (out_t, out_i, total.reshape(()), ind_t, ind_i)
